# baseline (speedup 1.0000x reference)
import jax
import jax.numpy as jnp
from jax import lax
from jax.experimental import pallas as pl
from jax.experimental.pallas import tpu as pltpu

_CH = 25
_B = 64
_C = 10647


def _fold8(t):
    parts = [t[i * 8:(i + 1) * 8] for i in range(4)]
    while len(parts) > 1:
        parts = [a + b for a, b in zip(parts[::2], parts[1::2])]
    return parts[0]


def _tc_body(x_ref, y_ref, o_ref, acc_ref, m_ref):
    ch = pl.program_id(0)
    rb = pl.program_id(1)

    xb = x_ref[0]
    yb = y_ref[0]

    @pl.when(ch == 0)
    def _():
        m_ref[rb] = jnp.where(yb > 0.5, 1.0, 0.0).astype(jnp.float32)

    m01 = m_ref[rb]
    d = yb - xb
    sq = d * d

    @pl.when(ch == 0)
    def _():
        t = _fold8(sq * m01 + (0.1 * (xb * xb)) * (1.0 - m01))

        @pl.when(rb == 0)
        def _():
            acc_ref[...] = t

        @pl.when(rb > 0)
        def _():
            acc_ref[...] += t

    @pl.when(ch > 0)
    def _():
        acc_ref[...] += _fold8(sq * m01)

    @pl.when((ch == _CH - 1) & (rb == 1))
    def _():
        o_ref[0] = jnp.sum(acc_ref[...])


_tc_call = pl.pallas_call(
    _tc_body,
    grid=(_CH, 2),
    in_specs=[
        pl.BlockSpec((1, _B // 2, _C), lambda ch, rb: (ch, rb, 0)),
        pl.BlockSpec((1, _B // 2, _C), lambda ch, rb: (ch, rb, 0)),
    ],
    out_specs=pl.BlockSpec(memory_space=pltpu.SMEM),
    out_shape=jax.ShapeDtypeStruct((1,), jnp.float32),
    scratch_shapes=[
        pltpu.VMEM((8, _C), jnp.float32),
        pltpu.VMEM((2, _B // 2, _C), jnp.float32),
    ],
)


def kernel(x, y):
    xt = jnp.transpose(x, (2, 0, 1))
    yt = jnp.transpose(y, (2, 0, 1))
    out = _tc_call(xt, yt)
    return out[0]


# speedup vs baseline: 1.2578x; 1.2578x over previous
import jax
import jax.numpy as jnp
from jax import lax
from jax.experimental import pallas as pl
from jax.experimental.pallas import tpu as pltpu

_CH = 25
_B = 64
_C = 10647


def _fold8(t):
    parts = [t[i * 8:(i + 1) * 8] for i in range(8)]
    while len(parts) > 1:
        parts = [a + b for a, b in zip(parts[::2], parts[1::2])]
    return parts[0]


def _tc_body(x_ref, y_ref, o_ref, acc_ref, m_ref):
    ch = pl.program_id(0)

    xb = x_ref[0]
    yb = y_ref[0]

    @pl.when(ch == 0)
    def _():
        m_ref[...] = jnp.where(yb > 0.5, 1.0, 0.0).astype(jnp.float32)

    m01 = m_ref[...]
    d = yb - xb
    sq = d * d

    @pl.when(ch == 0)
    def _():
        acc_ref[...] = _fold8(sq * m01 + (0.1 * (xb * xb)) * (1.0 - m01))

    @pl.when(ch > 0)
    def _():
        acc_ref[...] += _fold8(sq * m01)

    @pl.when(ch == _CH - 1)
    def _():
        o_ref[0] = jnp.sum(acc_ref[...])


_tc_call = pl.pallas_call(
    _tc_body,
    grid=(_CH,),
    in_specs=[
        pl.BlockSpec((1, _B, _C), lambda ch: (ch, 0, 0)),
        pl.BlockSpec((1, _B, _C), lambda ch: (ch, 0, 0)),
    ],
    out_specs=pl.BlockSpec(memory_space=pltpu.SMEM),
    out_shape=jax.ShapeDtypeStruct((1,), jnp.float32),
    scratch_shapes=[
        pltpu.VMEM((8, _C), jnp.float32),
        pltpu.VMEM((_B, _C), jnp.float32),
    ],
)


def kernel(x, y):
    xt = jnp.transpose(x, (2, 0, 1))
    yt = jnp.transpose(y, (2, 0, 1))
    out = _tc_call(xt, yt)
    return out[0]
